# baseline (speedup 1.0000x reference)
import functools

import jax
import jax.numpy as jnp
from jax import lax
from jax.experimental import pallas as pl
from jax.experimental.pallas import tpu as pltpu
from jax.experimental.pallas import tpu_sc as plsc

B = 16384
D = 64

_info = plsc.get_sparse_core_info()
_NC, _NS, _L = _info.num_cores, _info.num_subcores, _info.num_lanes
_NW = _NC * _NS
_BPW = B // _NW
_CHUNK = 128
_NCHUNK = _BPW // _CHUNK


def _sc_body(head_hbm, relidx_hbm, tail_hbm, ent_hbm, rel_emb_hbm,
             d_out, r_out,
             hidx, tidx, ridx, hrows, trows, rrows, hsem, tsem, rsem):
  wid = lax.axis_index("s") * _NC + lax.axis_index("c")
  base = wid * _BPW
  pltpu.sync_copy(head_hbm.at[wid], hidx)
  pltpu.sync_copy(tail_hbm.at[wid], tidx)
  pltpu.sync_copy(relidx_hbm.at[wid], ridx)
  rel_copies, ent_copies = [], []
  for k in range(_NCHUNK):
    sl = pl.ds(k * _CHUNK, _CHUNK)
    rel_copies.append(
        pltpu.async_copy(rel_emb_hbm.at[ridx.at[k]], rrows.at[sl], rsem))
    ent_copies.append(
        pltpu.async_copy(ent_hbm.at[hidx.at[k]], hrows.at[sl], hsem))
    ent_copies.append(
        pltpu.async_copy(ent_hbm.at[tidx.at[k]], trows.at[sl], tsem))
  for c in rel_copies:
    c.wait()
  pltpu.sync_copy(rrows, r_out.at[pl.ds(base, _BPW)])
  for c in ent_copies:
    c.wait()

  def sub_row(i, carry):
    for j in range(D // _L):
      sl = pl.ds(j * _L, _L)
      hrows[i, sl] = hrows[i, sl] - trows[i, sl]
    return carry

  lax.fori_loop(0, _BPW, sub_row, 0)
  pltpu.sync_copy(hrows, d_out.at[pl.ds(base, _BPW)])


_sc_gather = functools.partial(
    pl.kernel,
    mesh=plsc.VectorSubcoreMesh(core_axis_name="c", subcore_axis_name="s"),
    compiler_params=pltpu.CompilerParams(use_tc_tiling_on_sc=False),
    out_type=[jax.ShapeDtypeStruct((B, D), jnp.float32),
              jax.ShapeDtypeStruct((B, D), jnp.float32)],
    scratch_types=[
        pltpu.VMEM((_NCHUNK, _CHUNK), jnp.int32),
        pltpu.VMEM((_NCHUNK, _CHUNK), jnp.int32),
        pltpu.VMEM((_NCHUNK, _CHUNK), jnp.int32),
        pltpu.VMEM((_BPW, D), jnp.float32),
        pltpu.VMEM((_BPW, D), jnp.float32),
        pltpu.VMEM((_BPW, D), jnp.float32),
        pltpu.SemaphoreType.DMA,
        pltpu.SemaphoreType.DMA,
        pltpu.SemaphoreType.DMA,
    ],
)(_sc_body)


_BM = 1024


def _tc_body(d_ref, r_ref, t_ref, o_ref):
  o_ref[...] = (
      jnp.dot(d_ref[...], t_ref[...], preferred_element_type=jnp.float32)
      + r_ref[...])


def _tc_matmul(d, r, transfer):
  return pl.pallas_call(
      _tc_body,
      grid=(B // _BM,),
      in_specs=[
          pl.BlockSpec((_BM, D), lambda i: (i, 0)),
          pl.BlockSpec((_BM, D), lambda i: (i, 0)),
          pl.BlockSpec((D, D), lambda i: (0, 0)),
      ],
      out_specs=pl.BlockSpec((_BM, D), lambda i: (i, 0)),
      out_shape=jax.ShapeDtypeStruct((B, D), jnp.float32),
  )(d, r, transfer)


def kernel(head, relation, tail, ent_emb, rel_emb, transfer):
  head_r = head.astype(jnp.int32).reshape(_NW, _NCHUNK, _CHUNK)
  tail_r = tail.astype(jnp.int32).reshape(_NW, _NCHUNK, _CHUNK)
  rel_r = relation.astype(jnp.int32).reshape(_NW, _NCHUNK, _CHUNK)
  d, r = _sc_gather(head_r, rel_r, tail_r, ent_emb, rel_emb)
  return _tc_matmul(d, r, transfer)


# speedup vs baseline: 1.6154x; 1.6154x over previous
import functools

import jax
import jax.numpy as jnp
from jax import lax
from jax.experimental import pallas as pl
from jax.experimental.pallas import tpu as pltpu
from jax.experimental.pallas import tpu_sc as plsc

NUM_E = 1000000
NUM_R = 1000
B = 16384
D = 64

_info = plsc.get_sparse_core_info()
_NC, _NS, _L = _info.num_cores, _info.num_subcores, _info.num_lanes
_NW = _NC * _NS
_BPW = B // _NW
_CHUNK = 128
_NCHUNK = _BPW // _CHUNK

_BK = 2048
_K1_GRID = 245
_NE2 = _BK * _K1_GRID


def _k1_body(xlo_ref, xhi_ref, t_ref, o_ref):
  tb = t_ref[...].astype(jnp.bfloat16)
  dn = (((0,), (0,)), ((), ()))
  plo = lax.dot_general(xlo_ref[...].astype(jnp.bfloat16), tb, dn,
                        preferred_element_type=jnp.float32)
  phi = lax.dot_general(xhi_ref[...].astype(jnp.bfloat16), tb, dn,
                        preferred_element_type=jnp.float32)
  o_ref[...] = jnp.concatenate([plo, phi], axis=1)


_HI_MAX = (NUM_E - 1) // _BK


def _k1_proj(ent_t, transfer):
  return pl.pallas_call(
      _k1_body,
      grid=(_K1_GRID,),
      in_specs=[
          pl.BlockSpec((D, _BK), lambda i: (0, i)),
          pl.BlockSpec((D, _BK),
                       lambda i: (0, jnp.minimum(i + _K1_GRID, _HI_MAX))),
          pl.BlockSpec((D, D), lambda i: (0, 0)),
      ],
      out_specs=pl.BlockSpec((_BK, 2 * D), lambda i: (i, 0)),
      out_shape=jax.ShapeDtypeStruct((_NE2, 2 * D), jnp.float32),
  )(ent_t, ent_t, transfer)


def _k2_body(proj_hbm, rel2_hbm, qh_hbm, ph_hbm, qt_hbm, pt_hbm,
             qr_hbm, pr_hbm, out_hbm,
             qh, ph, qt, pt, qr, pr, h2, t2, r2, outv, hsem, tsem, rsem):
  wid = lax.axis_index("s") * _NC + lax.axis_index("c")
  base = wid * _BPW
  pltpu.sync_copy(qh_hbm.at[wid], qh)
  pltpu.sync_copy(ph_hbm.at[wid], ph)
  pltpu.sync_copy(qt_hbm.at[wid], qt)
  pltpu.sync_copy(pt_hbm.at[wid], pt)
  pltpu.sync_copy(qr_hbm.at[wid], qr)
  pltpu.sync_copy(pr_hbm.at[wid], pr)

  for k in range(_NCHUNK):
    ch = pltpu.async_copy(proj_hbm.at[qh.at[k]], h2, hsem)
    ct = pltpu.async_copy(proj_hbm.at[qt.at[k]], t2, tsem)
    cr = pltpu.async_copy(rel2_hbm.at[qr.at[k]], r2, rsem)
    ch.wait()
    ct.wait()
    cr.wait()
    for g in range(_CHUNK // _L):
      sl = pl.ds(g * _L, _L)
      rloc = lax.iota(jnp.int32, _L) + g * _L
      phv = ph[k, sl] * D
      ptv = pt[k, sl] * D
      prv = pr[k, sl] * D

      def dloop(d, carry, rloc=rloc, phv=phv, ptv=ptv, prv=prv, sl=sl):
        hv = plsc.load_gather(h2, [rloc, phv + d])
        tv = plsc.load_gather(t2, [rloc, ptv + d])
        rv = plsc.load_gather(r2, [rloc, prv + d])
        outv[d, sl] = hv - tv + rv
        return carry

      lax.fori_loop(0, D, dloop, 0)
    pltpu.sync_copy(
        outv,
        out_hbm.at[:, pl.ds(pl.multiple_of(base + k * _CHUNK, _CHUNK),
                            _CHUNK)])


_k2_gather = functools.partial(
    pl.kernel,
    mesh=plsc.VectorSubcoreMesh(core_axis_name="c", subcore_axis_name="s"),
    compiler_params=pltpu.CompilerParams(
        use_tc_tiling_on_sc=False, needs_layout_passes=False),
    out_type=jax.ShapeDtypeStruct((D, B), jnp.float32),
    scratch_types=[
        pltpu.VMEM((_NCHUNK, _CHUNK), jnp.int32),
        pltpu.VMEM((_NCHUNK, _CHUNK), jnp.int32),
        pltpu.VMEM((_NCHUNK, _CHUNK), jnp.int32),
        pltpu.VMEM((_NCHUNK, _CHUNK), jnp.int32),
        pltpu.VMEM((_NCHUNK, _CHUNK), jnp.int32),
        pltpu.VMEM((_NCHUNK, _CHUNK), jnp.int32),
        pltpu.VMEM((_CHUNK, 2 * D), jnp.float32),
        pltpu.VMEM((_CHUNK, 2 * D), jnp.float32),
        pltpu.VMEM((_CHUNK, 2 * D), jnp.float32),
        pltpu.VMEM((D, _CHUNK), jnp.float32),
        pltpu.SemaphoreType.DMA,
        pltpu.SemaphoreType.DMA,
        pltpu.SemaphoreType.DMA,
    ],
)(_k2_body)


def kernel(head, relation, tail, ent_emb, rel_emb, transfer):
  head = head.astype(jnp.int32)
  tail = tail.astype(jnp.int32)
  relation = relation.astype(jnp.int32)
  shape3 = (_NW, _NCHUNK, _CHUNK)
  qh = (head % _NE2).reshape(shape3)
  ph = (head // _NE2).reshape(shape3)
  qt = (tail % _NE2).reshape(shape3)
  pt = (tail // _NE2).reshape(shape3)
  nr2 = NUM_R // 2
  qr = (relation % nr2).reshape(shape3)
  pr = (relation // nr2).reshape(shape3)
  rel2 = jnp.concatenate([rel_emb[:nr2], rel_emb[nr2:]], axis=1)
  proj = _k1_proj(ent_emb.T, transfer)
  out_t = _k2_gather(proj, rel2, qh, ph, qt, pt, qr, pr)
  return out_t.T


# speedup vs baseline: 1.8841x; 1.1663x over previous
import functools

import jax
import jax.numpy as jnp
from jax import lax
from jax.experimental import pallas as pl
from jax.experimental.pallas import tpu as pltpu
from jax.experimental.pallas import tpu_sc as plsc

NUM_E = 1000000
NUM_R = 1000
B = 16384
D = 64

_info = plsc.get_sparse_core_info()
_NC, _NS, _L = _info.num_cores, _info.num_subcores, _info.num_lanes
_NW = _NC * _NS
_BPW = B // _NW
_CHUNK = 128
_NCHUNK = _BPW // _CHUNK

_BK = 2048
_K1_GRID = 245
_NE2 = _BK * _K1_GRID


def _k1_body(xlo_ref, xhi_ref, t_ref, o_ref):
  tb = t_ref[...].astype(jnp.bfloat16)
  dn = (((0,), (0,)), ((), ()))
  plo = lax.dot_general(xlo_ref[...].astype(jnp.bfloat16), tb, dn,
                        preferred_element_type=jnp.float32)
  phi = lax.dot_general(xhi_ref[...].astype(jnp.bfloat16), tb, dn,
                        preferred_element_type=jnp.float32)
  o_ref[...] = jnp.concatenate([plo, phi], axis=1)


_HI_MAX = (NUM_E - 1) // _BK


def _k1_proj(ent_t, transfer):
  return pl.pallas_call(
      _k1_body,
      grid=(_K1_GRID,),
      in_specs=[
          pl.BlockSpec((D, _BK), lambda i: (0, i)),
          pl.BlockSpec((D, _BK),
                       lambda i: (0, jnp.minimum(i + _K1_GRID, _HI_MAX))),
          pl.BlockSpec((D, D), lambda i: (0, 0)),
      ],
      out_specs=pl.BlockSpec((_BK, 2 * D), lambda i: (i, 0)),
      out_shape=jax.ShapeDtypeStruct((_NE2, 2 * D), jnp.float32),
  )(ent_t, ent_t, transfer)


def _k2_body(proj_hbm, rel_hbm, jh_hbm, jt_hbm, jr_hbm, out_hbm,
             jh, jt, jr, h2, t2, r2, ov, hsem, tsem, rsem):
  wid = lax.axis_index("s") * _NC + lax.axis_index("c")
  base = wid * _BPW
  pltpu.sync_copy(jh_hbm.at[wid], jh)
  pltpu.sync_copy(jt_hbm.at[wid], jt)
  pltpu.sync_copy(jr_hbm.at[wid], jr)

  for k in range(_NCHUNK):
    ch = pltpu.async_copy(proj_hbm.at[jh.at[k]], h2, hsem)
    ct = pltpu.async_copy(proj_hbm.at[jt.at[k]], t2, tsem)
    cr = pltpu.async_copy(rel_hbm.at[jr.at[k]], r2, rsem)
    ch.wait()
    ct.wait()
    cr.wait()

    def row(i, carry):
      for c in range(D // _L):
        sl = pl.ds(c * _L, _L)
        ov[i, sl] = h2[i, sl] - t2[i, sl] + r2[i, sl]
      return carry

    lax.fori_loop(0, _CHUNK, row, 0)
    pltpu.sync_copy(ov, out_hbm.at[pl.ds(base + k * _CHUNK, _CHUNK)])


_k2_gather = functools.partial(
    pl.kernel,
    mesh=plsc.VectorSubcoreMesh(core_axis_name="c", subcore_axis_name="s"),
    compiler_params=pltpu.CompilerParams(
        use_tc_tiling_on_sc=False, needs_layout_passes=False),
    out_type=jax.ShapeDtypeStruct((B, D), jnp.float32),
    scratch_types=[
        pltpu.VMEM((_NCHUNK, _CHUNK), jnp.int32),
        pltpu.VMEM((_NCHUNK, _CHUNK), jnp.int32),
        pltpu.VMEM((_NCHUNK, _CHUNK), jnp.int32),
        pltpu.VMEM((_CHUNK, D), jnp.float32),
        pltpu.VMEM((_CHUNK, D), jnp.float32),
        pltpu.VMEM((_CHUNK, D), jnp.float32),
        pltpu.VMEM((_CHUNK, D), jnp.float32),
        pltpu.SemaphoreType.DMA,
        pltpu.SemaphoreType.DMA,
        pltpu.SemaphoreType.DMA,
    ],
)(_k2_body)


def kernel(head, relation, tail, ent_emb, rel_emb, transfer):
  head = head.astype(jnp.int32)
  tail = tail.astype(jnp.int32)
  relation = relation.astype(jnp.int32)
  shape3 = (_NW, _NCHUNK, _CHUNK)
  jh = (2 * (head % _NE2) + head // _NE2).reshape(shape3)
  jt = (2 * (tail % _NE2) + tail // _NE2).reshape(shape3)
  nr2 = NUM_R // 2
  jr = (2 * (relation % nr2) + relation // nr2).reshape(shape3)
  rel2 = jnp.concatenate([rel_emb[:nr2], rel_emb[nr2:]],
                         axis=1).reshape(NUM_R, D)
  proj = _k1_proj(ent_emb.T, transfer).reshape(2 * _NE2, D)
  return _k2_gather(proj, rel2, jh, jt, jr)


# speedup vs baseline: 2.3386x; 1.2412x over previous
import functools

import jax
import jax.numpy as jnp
from jax import lax
from jax.experimental import pallas as pl
from jax.experimental.pallas import tpu as pltpu
from jax.experimental.pallas import tpu_sc as plsc

NUM_E = 1000000
NUM_R = 1000
B = 16384
D = 64

_info = plsc.get_sparse_core_info()
_NC, _NS, _L = _info.num_cores, _info.num_subcores, _info.num_lanes
_NW = _NC * _NS
_BPW = B // _NW
_CHUNK = 128
_NCHUNK = _BPW // _CHUNK

_BK = 2048
_K1_GRID = 123
_LQ = _BK * _K1_GRID
_HI_MAX = (NUM_E - 1) // _BK


def _k1_body(x0_ref, x1_ref, x2_ref, x3_ref, tlo_ref, thi_ref, o_ref):
  dn = (((0,), (0,)), ((), ()))
  tlo = tlo_ref[...].astype(jnp.bfloat16)
  thi = thi_ref[...].astype(jnp.bfloat16)
  los, his = [], []
  for xr in (x0_ref, x1_ref, x2_ref, x3_ref):
    xb = xr[...].astype(jnp.bfloat16)
    los.append(lax.dot_general(xb, tlo, dn,
                               preferred_element_type=jnp.float32))
    his.append(lax.dot_general(xb, thi, dn,
                               preferred_element_type=jnp.float32))
  lo = jnp.concatenate(los, axis=1)
  hi = jnp.concatenate(his, axis=1)
  lo16 = lax.bitcast_convert_type(lo.astype(jnp.bfloat16), jnp.uint16)
  hi16 = lax.bitcast_convert_type(hi.astype(jnp.bfloat16), jnp.uint16)
  packed = (hi16.astype(jnp.uint32) << 16) | lo16.astype(jnp.uint32)
  o_ref[...] = lax.bitcast_convert_type(packed, jnp.int32)


def _k1_proj(ent_t, tlo, thi):
  def xspec(p):
    return pl.BlockSpec(
        (D, _BK), lambda i, p=p: (0, jnp.minimum(i + p * _K1_GRID, _HI_MAX)))

  return pl.pallas_call(
      _k1_body,
      grid=(_K1_GRID,),
      in_specs=[xspec(0), xspec(1), xspec(2), xspec(3),
                pl.BlockSpec((D, D // 2), lambda i: (0, 0)),
                pl.BlockSpec((D, D // 2), lambda i: (0, 0))],
      out_specs=pl.BlockSpec((_BK, 2 * D), lambda i: (i, 0)),
      out_shape=jax.ShapeDtypeStruct((_LQ, 2 * D), jnp.int32),
  )(ent_t, ent_t, ent_t, ent_t, tlo, thi)


def _k2_body(proj_hbm, rel_hbm, jh_hbm, jt_hbm, jr_hbm, out_hbm,
             jh, jt, jr, h2, t2, r2, ov, hsem, tsem, rsem):
  wid = lax.axis_index("s") * _NC + lax.axis_index("c")
  base = wid * _BPW
  pltpu.sync_copy(jh_hbm.at[wid], jh)
  pltpu.sync_copy(jt_hbm.at[wid], jt)
  pltpu.sync_copy(jr_hbm.at[wid], jr)

  nw = D // 2

  for k in range(_NCHUNK):
    ch = pltpu.async_copy(proj_hbm.at[jh.at[k]], h2, hsem)
    ct = pltpu.async_copy(proj_hbm.at[jt.at[k]], t2, tsem)
    cr = pltpu.async_copy(rel_hbm.at[jr.at[k]], r2, rsem)
    ch.wait()
    ct.wait()
    cr.wait()

    def row(i, carry):
      for c in range(nw // _L):
        sl = pl.ds(c * _L, _L)
        hv = plsc.bitcast(h2[i, sl], jnp.bfloat16)
        tv = plsc.bitcast(t2[i, sl], jnp.bfloat16)
        rv = plsc.bitcast(r2[i, sl], jnp.bfloat16)
        ov[i, pl.ds(c * 2 * _L, 2 * _L)] = hv - tv + rv
      return carry

    lax.fori_loop(0, _CHUNK, row, 0)
    pltpu.sync_copy(ov, out_hbm.at[pl.ds(base + k * _CHUNK, _CHUNK)])


_k2_gather = functools.partial(
    pl.kernel,
    mesh=plsc.VectorSubcoreMesh(core_axis_name="c", subcore_axis_name="s"),
    compiler_params=pltpu.CompilerParams(
        use_tc_tiling_on_sc=False, needs_layout_passes=False),
    out_type=jax.ShapeDtypeStruct((B, D), jnp.bfloat16),
    scratch_types=[
        pltpu.VMEM((_NCHUNK, _CHUNK), jnp.int32),
        pltpu.VMEM((_NCHUNK, _CHUNK), jnp.int32),
        pltpu.VMEM((_NCHUNK, _CHUNK), jnp.int32),
        pltpu.VMEM((_CHUNK, D // 2), jnp.int32),
        pltpu.VMEM((_CHUNK, D // 2), jnp.int32),
        pltpu.VMEM((_CHUNK, D // 2), jnp.int32),
        pltpu.VMEM((_CHUNK, D), jnp.bfloat16),
        pltpu.SemaphoreType.DMA,
        pltpu.SemaphoreType.DMA,
        pltpu.SemaphoreType.DMA,
    ],
)(_k2_body)


def kernel(head, relation, tail, ent_emb, rel_emb, transfer):
  head = head.astype(jnp.int32)
  tail = tail.astype(jnp.int32)
  relation = relation.astype(jnp.int32)
  shape3 = (_NW, _NCHUNK, _CHUNK)
  jh = (4 * (head % _LQ) + head // _LQ).reshape(shape3)
  jt = (4 * (tail % _LQ) + tail // _LQ).reshape(shape3)
  jr = relation.reshape(shape3)
  tlo = transfer[:, 0::2]
  thi = transfer[:, 1::2]
  rel_i32 = lax.bitcast_convert_type(
      rel_emb.astype(jnp.bfloat16).reshape(NUM_R, D // 2, 2), jnp.int32)
  proj = _k1_proj(ent_emb.T, tlo, thi).reshape(4 * _LQ, D // 2)
  out16 = _k2_gather(proj, rel_i32, jh, jt, jr)
  return out16.astype(jnp.float32)


# speedup vs baseline: 2.4990x; 1.0686x over previous
import functools

import jax
import jax.numpy as jnp
from jax import lax
from jax.experimental import pallas as pl
from jax.experimental.pallas import tpu as pltpu
from jax.experimental.pallas import tpu_sc as plsc

NUM_E = 1000000
NUM_R = 1000
B = 16384
D = 64

_info = plsc.get_sparse_core_info()
_NC, _NS, _L = _info.num_cores, _info.num_subcores, _info.num_lanes
_NW = _NC * _NS
_BPW = B // _NW
_CHUNK = 128
_NCHUNK = _BPW // _CHUNK

_BK = 8192
_K1_GRID = 32
_LQ = _BK * _K1_GRID
_HI_MAX = (NUM_E - 1) // _BK


def _k1_body(x0_ref, x1_ref, x2_ref, x3_ref, tlo_ref, thi_ref, o_ref):
  dn = (((0,), (0,)), ((), ()))
  tlo = tlo_ref[...].astype(jnp.bfloat16)
  thi = thi_ref[...].astype(jnp.bfloat16)
  los, his = [], []
  for xr in (x0_ref, x1_ref, x2_ref, x3_ref):
    xb = xr[...].astype(jnp.bfloat16)
    los.append(lax.dot_general(xb, tlo, dn,
                               preferred_element_type=jnp.float32))
    his.append(lax.dot_general(xb, thi, dn,
                               preferred_element_type=jnp.float32))
  lo = jnp.concatenate(los, axis=1)
  hi = jnp.concatenate(his, axis=1)
  lo16 = lax.bitcast_convert_type(lo.astype(jnp.bfloat16), jnp.uint16)
  hi16 = lax.bitcast_convert_type(hi.astype(jnp.bfloat16), jnp.uint16)
  packed = (hi16.astype(jnp.uint32) << 16) | lo16.astype(jnp.uint32)
  o_ref[...] = lax.bitcast_convert_type(packed, jnp.int32)


def _k1_proj(ent_t, tlo, thi):
  def xspec(p):
    return pl.BlockSpec(
        (D, _BK), lambda i, p=p: (0, jnp.minimum(i + p * _K1_GRID, _HI_MAX)))

  return pl.pallas_call(
      _k1_body,
      grid=(_K1_GRID,),
      in_specs=[xspec(0), xspec(1), xspec(2), xspec(3),
                pl.BlockSpec((D, D // 2), lambda i: (0, 0)),
                pl.BlockSpec((D, D // 2), lambda i: (0, 0))],
      out_specs=pl.BlockSpec((_BK, 2 * D), lambda i: (i, 0)),
      out_shape=jax.ShapeDtypeStruct((_LQ, 2 * D), jnp.int32),
  )(ent_t, ent_t, ent_t, ent_t, tlo, thi)


def _k2_body(proj_hbm, rel_hbm, jh_hbm, jt_hbm, jr_hbm, out_hbm,
             jh, jt, jr, h2, t2, r2, ov, hsem, tsem, rsem):
  wid = lax.axis_index("s") * _NC + lax.axis_index("c")
  base = wid * _BPW
  pltpu.sync_copy(jh_hbm.at[wid], jh)
  pltpu.sync_copy(jt_hbm.at[wid], jt)
  pltpu.sync_copy(jr_hbm.at[wid], jr)

  nw = D // 2

  for k in range(_NCHUNK):
    ch = pltpu.async_copy(proj_hbm.at[jh.at[k]], h2, hsem)
    ct = pltpu.async_copy(proj_hbm.at[jt.at[k]], t2, tsem)
    cr = pltpu.async_copy(rel_hbm.at[jr.at[k]], r2, rsem)
    ch.wait()
    ct.wait()
    cr.wait()

    def row(i, carry):
      for c in range(nw // _L):
        sl = pl.ds(c * _L, _L)
        hv = plsc.bitcast(h2[i, sl], jnp.bfloat16)
        tv = plsc.bitcast(t2[i, sl], jnp.bfloat16)
        rv = plsc.bitcast(r2[i, sl], jnp.bfloat16)
        ov[i, pl.ds(c * 2 * _L, 2 * _L)] = hv - tv + rv
      return carry

    lax.fori_loop(0, _CHUNK, row, 0)
    pltpu.sync_copy(ov, out_hbm.at[pl.ds(base + k * _CHUNK, _CHUNK)])


_k2_gather = functools.partial(
    pl.kernel,
    mesh=plsc.VectorSubcoreMesh(core_axis_name="c", subcore_axis_name="s"),
    compiler_params=pltpu.CompilerParams(
        use_tc_tiling_on_sc=False, needs_layout_passes=False),
    out_type=jax.ShapeDtypeStruct((B, D), jnp.bfloat16),
    scratch_types=[
        pltpu.VMEM((_NCHUNK, _CHUNK), jnp.int32),
        pltpu.VMEM((_NCHUNK, _CHUNK), jnp.int32),
        pltpu.VMEM((_NCHUNK, _CHUNK), jnp.int32),
        pltpu.VMEM((_CHUNK, D // 2), jnp.int32),
        pltpu.VMEM((_CHUNK, D // 2), jnp.int32),
        pltpu.VMEM((_CHUNK, D // 2), jnp.int32),
        pltpu.VMEM((_CHUNK, D), jnp.bfloat16),
        pltpu.SemaphoreType.DMA,
        pltpu.SemaphoreType.DMA,
        pltpu.SemaphoreType.DMA,
    ],
)(_k2_body)


def kernel(head, relation, tail, ent_emb, rel_emb, transfer):
  head = head.astype(jnp.int32)
  tail = tail.astype(jnp.int32)
  relation = relation.astype(jnp.int32)
  shape3 = (_NW, _NCHUNK, _CHUNK)
  jh = (4 * (head & (_LQ - 1)) + (head >> 18)).reshape(shape3)
  jt = (4 * (tail & (_LQ - 1)) + (tail >> 18)).reshape(shape3)
  jr = relation.reshape(shape3)
  tlo = transfer[:, 0::2]
  thi = transfer[:, 1::2]
  rel_i32 = lax.bitcast_convert_type(
      rel_emb.astype(jnp.bfloat16).reshape(NUM_R, D // 2, 2), jnp.int32)
  proj = _k1_proj(ent_emb.T, tlo, thi).reshape(4 * _LQ, D // 2)
  out16 = _k2_gather(proj, rel_i32, jh, jt, jr)
  return out16.astype(jnp.float32)


# speedup vs baseline: 2.5127x; 1.0055x over previous
import functools

import jax
import jax.numpy as jnp
from jax import lax
from jax.experimental import pallas as pl
from jax.experimental.pallas import tpu as pltpu
from jax.experimental.pallas import tpu_sc as plsc

NUM_E = 1000000
NUM_R = 1000
B = 16384
D = 64

_info = plsc.get_sparse_core_info()
_NC, _NS, _L = _info.num_cores, _info.num_subcores, _info.num_lanes
_NW = _NC * _NS
_BPW = B // _NW
_CHUNK = 128
_NCHUNK = _BPW // _CHUNK

_BK = 8192
_K1_GRID = 32
_LQ = _BK * _K1_GRID
_HI_MAX = (NUM_E - 1) // _BK


def _k1_body(x0_ref, x1_ref, x2_ref, x3_ref, tlo_ref, thi_ref, o_ref):
  dn = (((0,), (0,)), ((), ()))
  tlo = tlo_ref[...].astype(jnp.bfloat16)
  thi = thi_ref[...].astype(jnp.bfloat16)
  los, his = [], []
  for xr in (x0_ref, x1_ref, x2_ref, x3_ref):
    xb = xr[...].astype(jnp.bfloat16)
    los.append(lax.dot_general(xb, tlo, dn,
                               preferred_element_type=jnp.float32))
    his.append(lax.dot_general(xb, thi, dn,
                               preferred_element_type=jnp.float32))
  lo = jnp.concatenate(los, axis=1)
  hi = jnp.concatenate(his, axis=1)
  lo16 = lax.bitcast_convert_type(lo.astype(jnp.bfloat16), jnp.uint16)
  hi16 = lax.bitcast_convert_type(hi.astype(jnp.bfloat16), jnp.uint16)
  packed = (hi16.astype(jnp.uint32) << 16) | lo16.astype(jnp.uint32)
  o_ref[...] = lax.bitcast_convert_type(packed, jnp.int32)


def _k1_proj(ent_t, tlo, thi):
  def xspec(p):
    return pl.BlockSpec(
        (D, _BK), lambda i, p=p: (0, jnp.minimum(i + p * _K1_GRID, _HI_MAX)))

  return pl.pallas_call(
      _k1_body,
      grid=(_K1_GRID,),
      in_specs=[xspec(0), xspec(1), xspec(2), xspec(3),
                pl.BlockSpec((D, D // 2), lambda i: (0, 0)),
                pl.BlockSpec((D, D // 2), lambda i: (0, 0))],
      out_specs=pl.BlockSpec((_BK, 2 * D), lambda i: (i, 0)),
      out_shape=jax.ShapeDtypeStruct((_LQ, 2 * D), jnp.int32),
  )(ent_t, ent_t, ent_t, ent_t, tlo, thi)


def _k2_body(proj_hbm, rel_hbm, jh_hbm, jt_hbm, jr_hbm, out_hbm,
             jh, jt, jr, h2, t2, r2, ov, hsem, tsem, rsem):
  wid = lax.axis_index("s") * _NC + lax.axis_index("c")
  base = wid * _BPW
  pltpu.sync_copy(jh_hbm.at[wid], jh)
  pltpu.sync_copy(jt_hbm.at[wid], jt)
  pltpu.sync_copy(jr_hbm.at[wid], jr)

  nw = D // 2

  def fire(k, buf):
    return (pltpu.async_copy(proj_hbm.at[jh.at[k]], h2.at[buf], hsem),
            pltpu.async_copy(proj_hbm.at[jt.at[k]], t2.at[buf], tsem),
            pltpu.async_copy(rel_hbm.at[jr.at[k]], r2.at[buf], rsem))

  pend = fire(0, 0)
  for k in range(_NCHUNK):
    nxt = fire(k + 1, (k + 1) % 2) if k + 1 < _NCHUNK else None
    for c in pend:
      c.wait()
    buf = k % 2

    def row(i, carry, buf=buf):
      for c in range(nw // _L):
        sl = pl.ds(c * _L, _L)
        hv = plsc.bitcast(h2[buf, i, sl], jnp.bfloat16)
        tv = plsc.bitcast(t2[buf, i, sl], jnp.bfloat16)
        rv = plsc.bitcast(r2[buf, i, sl], jnp.bfloat16)
        ov[i, pl.ds(c * 2 * _L, 2 * _L)] = hv - tv + rv
      return carry

    lax.fori_loop(0, _CHUNK, row, 0)
    pltpu.sync_copy(ov, out_hbm.at[pl.ds(base + k * _CHUNK, _CHUNK)])
    pend = nxt


_k2_gather = functools.partial(
    pl.kernel,
    mesh=plsc.VectorSubcoreMesh(core_axis_name="c", subcore_axis_name="s"),
    compiler_params=pltpu.CompilerParams(
        use_tc_tiling_on_sc=False, needs_layout_passes=False),
    out_type=jax.ShapeDtypeStruct((B, D), jnp.bfloat16),
    scratch_types=[
        pltpu.VMEM((_NCHUNK, _CHUNK), jnp.int32),
        pltpu.VMEM((_NCHUNK, _CHUNK), jnp.int32),
        pltpu.VMEM((_NCHUNK, _CHUNK), jnp.int32),
        pltpu.VMEM((2, _CHUNK, D // 2), jnp.int32),
        pltpu.VMEM((2, _CHUNK, D // 2), jnp.int32),
        pltpu.VMEM((2, _CHUNK, D // 2), jnp.int32),
        pltpu.VMEM((_CHUNK, D), jnp.bfloat16),
        pltpu.SemaphoreType.DMA,
        pltpu.SemaphoreType.DMA,
        pltpu.SemaphoreType.DMA,
    ],
)(_k2_body)


def kernel(head, relation, tail, ent_emb, rel_emb, transfer):
  head = head.astype(jnp.int32)
  tail = tail.astype(jnp.int32)
  relation = relation.astype(jnp.int32)
  shape3 = (_NW, _NCHUNK, _CHUNK)
  jh = (4 * (head & (_LQ - 1)) + (head >> 18)).reshape(shape3)
  jt = (4 * (tail & (_LQ - 1)) + (tail >> 18)).reshape(shape3)
  jr = relation.reshape(shape3)
  tlo = transfer[:, 0::2]
  thi = transfer[:, 1::2]
  rel_i32 = lax.bitcast_convert_type(
      rel_emb.astype(jnp.bfloat16).reshape(NUM_R, D // 2, 2), jnp.int32)
  proj = _k1_proj(ent_emb.T, tlo, thi).reshape(4 * _LQ, D // 2)
  out16 = _k2_gather(proj, rel_i32, jh, jt, jr)
  return out16.astype(jnp.float32)
